# Initial kernel scaffold; baseline (speedup 1.0000x reference)
import jax
import jax.numpy as jnp
from jax.experimental import pallas as pl


def kernel(preprocessed_tags, embedding_weight):
    raise NotImplementedError("write your pallas kernel here")


# speedup vs baseline: 2.9341x; 2.9341x over previous
import functools

import jax
import jax.numpy as jnp
from jax import lax
from jax.experimental import pallas as pl
from jax.experimental.pallas import tpu as pltpu
from jax.experimental.pallas import tpu_sc as plsc

K = 100000
D = 64
B = 4096 * 50

_info = plsc.get_sparse_core_info()
NC, NS, L = _info.num_cores, _info.num_subcores, _info.num_lanes
NW = NC * NS
B_PER_W = B // NW
CHUNK = 128
N_CHUNKS = B_PER_W // CHUNK


def _body(idx_hbm, table_hbm, out_hbm, idx_v, rows_v, sem):
    wid = lax.axis_index("s") * NC + lax.axis_index("c")
    base = wid * B_PER_W

    pltpu.sync_copy(idx_hbm.at[pl.ds(base, B_PER_W)], idx_v)

    def chunk_body(c, carry):
        off = c * CHUNK
        pltpu.async_copy(
            table_hbm.at[idx_v.at[pl.ds(off, CHUNK)]], rows_v, sem
        ).wait()

        def relu_row(i, c2):
            for j in range(D // L):
                s = pl.ds(j * L, L)
                rows_v[i, s] = jnp.maximum(rows_v[i, s], 0.0)
            return c2

        lax.fori_loop(0, CHUNK, relu_row, 0)

        pltpu.sync_copy(rows_v, out_hbm.at[pl.ds(base + off, CHUNK)])
        return carry

    lax.fori_loop(0, N_CHUNKS, chunk_body, 0)


@jax.jit
def _run(idx, table):
    mesh = plsc.VectorSubcoreMesh(core_axis_name="c", subcore_axis_name="s")
    return pl.kernel(
        _body,
        out_type=jax.ShapeDtypeStruct((B, D), jnp.float32),
        mesh=mesh,
        scratch_types=[
            pltpu.VMEM((B_PER_W,), jnp.int32),
            pltpu.VMEM((CHUNK, D), jnp.float32),
            pltpu.SemaphoreType.DMA,
        ],
        compiler_params=pltpu.CompilerParams(use_tc_tiling_on_sc=False),
    )(idx, table)


def kernel(preprocessed_tags, embedding_weight):
    idx = preprocessed_tags.reshape(-1).astype(jnp.int32)
    out = _run(idx, embedding_weight)
    return out.reshape(preprocessed_tags.shape + (D,))


# speedup vs baseline: 3.3557x; 1.1437x over previous
import jax
import jax.numpy as jnp
from jax import lax
from jax.experimental import pallas as pl
from jax.experimental.pallas import tpu as pltpu
from jax.experimental.pallas import tpu_sc as plsc

K = 100000
D = 64
B = 4096 * 50

_info = plsc.get_sparse_core_info()
NC, NS, L = _info.num_cores, _info.num_subcores, _info.num_lanes
NW = NC * NS
B_PER_W = B // NW
CHUNK = 128
N_CHUNKS = B_PER_W // CHUNK


def _body(idx_hbm, table_hbm, out_hbm, idx_v, rows0, rows1, gsem, ssem):
    wid = lax.axis_index("s") * NC + lax.axis_index("c")
    base = wid * B_PER_W

    pltpu.sync_copy(idx_hbm.at[pl.ds(base, B_PER_W)], idx_v)

    bufs = (rows0, rows1)

    def sg(c, buf):
        off = pl.multiple_of(c * CHUNK, CHUNK)
        pltpu.async_copy(table_hbm.at[idx_v.at[pl.ds(off, CHUNK)]], buf, gsem)

    def ss(c, buf):
        off = pl.multiple_of(base + c * CHUNK, CHUNK)
        pltpu.async_copy(buf, out_hbm.at[pl.ds(off, CHUNK)], ssem)

    def wg(buf):
        pltpu.make_async_copy(table_hbm.at[pl.ds(0, CHUNK)], buf, gsem).wait()

    def ws(buf):
        pltpu.make_async_copy(buf, out_hbm.at[pl.ds(base, CHUNK)], ssem).wait()

    def relu(buf):
        @plsc.parallel_loop(0, CHUNK, step=4)
        def _relu_rows(i):
            for r in range(4):
                for j in range(D // L):
                    s = pl.ds(j * L, L)
                    buf[i + r, s] = jnp.maximum(buf[i + r, s], 0.0)

    sg(0, rows0)
    wg(rows0)
    sg(1, rows1)
    relu(rows0)
    ss(0, rows0)

    def outer(t, carry):
        for b2 in range(2):
            bi = (1 + b2) % 2
            buf, nbuf = bufs[bi], bufs[1 - bi]
            c = 1 + 2 * t + b2
            wg(buf)
            ws(nbuf)
            sg(c + 1, nbuf)
            relu(buf)
            ss(c, buf)
        return carry

    lax.fori_loop(0, (N_CHUNKS - 2) // 2, outer, 0)

    wg(rows1)
    ws(rows0)
    relu(rows1)
    ss(N_CHUNKS - 1, rows1)
    ws(rows1)


@jax.jit
def _run(idx, table):
    mesh = plsc.VectorSubcoreMesh(core_axis_name="c", subcore_axis_name="s")
    return pl.kernel(
        _body,
        out_type=jax.ShapeDtypeStruct((B, D), jnp.float32),
        mesh=mesh,
        scratch_types=[
            pltpu.VMEM((B_PER_W,), jnp.int32),
            pltpu.VMEM((CHUNK, D), jnp.float32),
            pltpu.VMEM((CHUNK, D), jnp.float32),
            pltpu.SemaphoreType.DMA,
            pltpu.SemaphoreType.DMA,
        ],
        compiler_params=pltpu.CompilerParams(use_tc_tiling_on_sc=False),
    )(idx, table)


def kernel(preprocessed_tags, embedding_weight):
    idx = preprocessed_tags.reshape(-1).astype(jnp.int32)
    out = _run(idx, embedding_weight)
    return out.reshape(preprocessed_tags.shape + (D,))


# speedup vs baseline: 3.6468x; 1.0868x over previous
import jax
import jax.numpy as jnp
from jax import lax
from jax.experimental import pallas as pl
from jax.experimental.pallas import tpu as pltpu
from jax.experimental.pallas import tpu_sc as plsc

K = 100000
D = 64
NSENT = 4096
LS = 50
B = NSENT * LS

_info = plsc.get_sparse_core_info()
NC, NS, L = _info.num_cores, _info.num_subcores, _info.num_lanes
NW = NC * NS
S_PER_W = NSENT // NW
SENT_PER_SLOT = 4
ROWS = SENT_PER_SLOT * LS
N_SLOTS = S_PER_W // SENT_PER_SLOT
NBUF = 4
AHEAD = 2


def _body(idx_hbm, table_hbm, out_hbm, idx_v, b0, b1, b2, b3,
          g0, g1, g2, g3, s0, s1, s2, s3):
    wid = lax.axis_index("s") * NC + lax.axis_index("c")
    sent0 = wid * S_PER_W
    base = sent0 * LS

    pltpu.sync_copy(idx_hbm.at[pl.ds(sent0, S_PER_W)], idx_v)

    bufs = (b0, b1, b2, b3)
    gsems = (g0, g1, g2, g3)
    ssems = (s0, s1, s2, s3)

    def sg(t, b):
        for j in range(SENT_PER_SLOT):
            pltpu.async_copy(
                table_hbm.at[idx_v.at[t * SENT_PER_SLOT + j]],
                bufs[b].at[pl.ds(j * LS, LS)],
                gsems[b],
            )

    def wg(b):
        for j in range(SENT_PER_SLOT):
            pltpu.make_async_copy(
                table_hbm.at[pl.ds(0, LS)],
                bufs[b].at[pl.ds(j * LS, LS)],
                gsems[b],
            ).wait()

    def ss(t, b):
        pltpu.async_copy(
            bufs[b], out_hbm.at[pl.ds(base + t * ROWS, ROWS)], ssems[b]
        )

    def ws(b):
        pltpu.make_async_copy(
            bufs[b], out_hbm.at[pl.ds(base, ROWS)], ssems[b]
        ).wait()

    def relu(b):
        buf = bufs[b]

        @plsc.parallel_loop(0, ROWS, step=4)
        def _relu_rows(i):
            for r in range(4):
                for j in range(D // L):
                    s = pl.ds(j * L, L)
                    buf[i + r, s] = jnp.maximum(buf[i + r, s], 0.0)

    sg(0, 0)
    sg(1, 1)
    sg(2, 2)
    wg(0)
    relu(0)
    ss(0, 0)
    sg(3, 3)
    wg(1)
    relu(1)
    ss(1, 1)

    def outer(k, carry):
        for j in range(4):
            t = 2 + k * 4 + j
            b = (2 + j) % NBUF
            nb = (b + AHEAD) % NBUF
            ws(nb)
            sg(t + AHEAD, nb)
            wg(b)
            relu(b)
            ss(t, b)
        return carry

    lax.fori_loop(0, (N_SLOTS - 4) // 4, outer, 0)

    for t in (N_SLOTS - 2, N_SLOTS - 1):
        b = t % NBUF
        ws((b + 2) % NBUF)
        wg(b)
        relu(b)
        ss(t, b)
    ws((N_SLOTS - 2) % NBUF)
    ws((N_SLOTS - 1) % NBUF)


@jax.jit
def _run(tags, table):
    mesh = plsc.VectorSubcoreMesh(core_axis_name="c", subcore_axis_name="s")
    return pl.kernel(
        _body,
        out_type=jax.ShapeDtypeStruct((B, D), jnp.float32),
        mesh=mesh,
        scratch_types=[
            pltpu.VMEM((S_PER_W, LS), jnp.int32),
            pltpu.VMEM((ROWS, D), jnp.float32),
            pltpu.VMEM((ROWS, D), jnp.float32),
            pltpu.VMEM((ROWS, D), jnp.float32),
            pltpu.VMEM((ROWS, D), jnp.float32),
            pltpu.SemaphoreType.DMA,
            pltpu.SemaphoreType.DMA,
            pltpu.SemaphoreType.DMA,
            pltpu.SemaphoreType.DMA,
            pltpu.SemaphoreType.DMA,
            pltpu.SemaphoreType.DMA,
            pltpu.SemaphoreType.DMA,
            pltpu.SemaphoreType.DMA,
        ],
        compiler_params=pltpu.CompilerParams(use_tc_tiling_on_sc=False),
    )(tags, table)


def kernel(preprocessed_tags, embedding_weight):
    tags = preprocessed_tags.astype(jnp.int32)
    out = _run(tags, embedding_weight)
    return out.reshape(preprocessed_tags.shape + (D,))


# speedup vs baseline: 3.6495x; 1.0007x over previous
import jax
import jax.numpy as jnp
from jax import lax
from jax.experimental import pallas as pl
from jax.experimental.pallas import tpu as pltpu
from jax.experimental.pallas import tpu_sc as plsc

K = 100000
D = 64
NSENT = 4096
LS = 50
B = NSENT * LS

_info = plsc.get_sparse_core_info()
NC, NS, L = _info.num_cores, _info.num_subcores, _info.num_lanes
NW = NC * NS
S_PER_W = NSENT // NW
SENT_PER_SLOT = 4
ROWS = SENT_PER_SLOT * LS
N_SLOTS = S_PER_W // SENT_PER_SLOT
NBUF = 4
AHEAD = 2


def _body(idx_hbm, table_hbm, out_hbm, idx_v, b0, b1, b2, b3,
          g0, g1, g2, g3, s0, s1, s2, s3):
    wid = lax.axis_index("s") * NC + lax.axis_index("c")
    sent0 = wid * S_PER_W

    pltpu.sync_copy(idx_hbm.at[pl.ds(sent0, S_PER_W)], idx_v)

    bufs = (b0, b1, b2, b3)
    gsems = (g0, g1, g2, g3)
    ssems = (s0, s1, s2, s3)

    def sg(t, b):
        for j in range(SENT_PER_SLOT):
            pltpu.async_copy(
                table_hbm.at[idx_v.at[t * SENT_PER_SLOT + j]],
                bufs[b].at[j],
                gsems[b],
            )

    def wg(b):
        for j in range(SENT_PER_SLOT):
            pltpu.make_async_copy(
                table_hbm.at[pl.ds(0, LS)],
                bufs[b].at[j],
                gsems[b],
            ).wait()

    def ss(t, b):
        pltpu.async_copy(
            bufs[b],
            out_hbm.at[pl.ds(sent0 + t * SENT_PER_SLOT, SENT_PER_SLOT)],
            ssems[b],
        )

    def ws(b):
        pltpu.make_async_copy(
            bufs[b],
            out_hbm.at[pl.ds(sent0, SENT_PER_SLOT)],
            ssems[b],
        ).wait()

    def relu(b):
        buf = bufs[b]

        @plsc.parallel_loop(0, LS, step=2)
        def _relu_rows(i):
            for j in range(SENT_PER_SLOT):
                for r in range(2):
                    for k in range(D // L):
                        s = pl.ds(k * L, L)
                        buf[j, i + r, s] = jnp.maximum(buf[j, i + r, s], 0.0)

    sg(0, 0)
    sg(1, 1)
    sg(2, 2)
    wg(0)
    relu(0)
    ss(0, 0)
    sg(3, 3)
    wg(1)
    relu(1)
    ss(1, 1)

    def outer(k, carry):
        for j in range(4):
            t = 2 + k * 4 + j
            b = (2 + j) % NBUF
            nb = (b + AHEAD) % NBUF
            ws(nb)
            sg(t + AHEAD, nb)
            wg(b)
            relu(b)
            ss(t, b)
        return carry

    lax.fori_loop(0, (N_SLOTS - 4) // 4, outer, 0)

    for t in (N_SLOTS - 2, N_SLOTS - 1):
        b = t % NBUF
        ws((b + 2) % NBUF)
        wg(b)
        relu(b)
        ss(t, b)
    ws((N_SLOTS - 2) % NBUF)
    ws((N_SLOTS - 1) % NBUF)


@jax.jit
def _run(tags, table):
    mesh = plsc.VectorSubcoreMesh(core_axis_name="c", subcore_axis_name="s")
    return pl.kernel(
        _body,
        out_type=jax.ShapeDtypeStruct((NSENT, LS, D), jnp.float32),
        mesh=mesh,
        scratch_types=[
            pltpu.VMEM((S_PER_W, LS), jnp.int32),
            pltpu.VMEM((SENT_PER_SLOT, LS, D), jnp.float32),
            pltpu.VMEM((SENT_PER_SLOT, LS, D), jnp.float32),
            pltpu.VMEM((SENT_PER_SLOT, LS, D), jnp.float32),
            pltpu.VMEM((SENT_PER_SLOT, LS, D), jnp.float32),
            pltpu.SemaphoreType.DMA,
            pltpu.SemaphoreType.DMA,
            pltpu.SemaphoreType.DMA,
            pltpu.SemaphoreType.DMA,
            pltpu.SemaphoreType.DMA,
            pltpu.SemaphoreType.DMA,
            pltpu.SemaphoreType.DMA,
            pltpu.SemaphoreType.DMA,
        ],
        compiler_params=pltpu.CompilerParams(use_tc_tiling_on_sc=False),
    )(tags, table)


def kernel(preprocessed_tags, embedding_weight):
    tags = preprocessed_tags.astype(jnp.int32)
    return _run(tags, embedding_weight)
